# Initial kernel scaffold; baseline (speedup 1.0000x reference)
import jax
import jax.numpy as jnp
from jax.experimental import pallas as pl


def kernel(x, edge_index, edge_attr, W, att_src, att_dst, W_edge, att_edge, bias, We, be, gamma, beta):
    raise NotImplementedError("write your pallas kernel here")


# speedup vs baseline: 13.6926x; 13.6926x over previous
import functools

import jax
import jax.numpy as jnp
from jax import lax
from jax.experimental import pallas as pl
from jax.experimental.pallas import tpu as pltpu
from jax.experimental.pallas import tpu_sc as plsc

N = 10000
E = 320000
F = 128
ED = 16

NC = 2
NS = 16
L = 16

B = 80
SB = 25
NSB = E // (SB * B)
SPT = NSB // NS
SPW = NSB // (NC * NS)
NRC = N // B


def _node_body(x_ref, w_ref, av_ref, h_ref, a2_ref):
    hb = jnp.dot(x_ref[...], w_ref[...], preferred_element_type=jnp.float32)
    h_ref[...] = hb
    a2_ref[...] = jnp.dot(hb, av_ref[...], preferred_element_type=jnp.float32)


def _tc_node(x, W, av2):
    br = 200
    grid = N // br
    return pl.pallas_call(
        _node_body,
        grid=(grid,),
        in_specs=[
            pl.BlockSpec((br, F), lambda i: (i, 0)),
            pl.BlockSpec((F, F), lambda i: (0, 0)),
            pl.BlockSpec((F, 2), lambda i: (0, 0)),
        ],
        out_specs=[
            pl.BlockSpec((br, F), lambda i: (i, 0)),
            pl.BlockSpec((br, 2), lambda i: (i, 0)),
        ],
        out_shape=[
            jax.ShapeDtypeStruct((N, F), jnp.float32),
            jax.ShapeDtypeStruct((N, 2), jnp.float32),
        ],
    )(x, W, av2)


def _edge_body(ea_ref, we_ref, be_ref, wedge_ref, attedge_ref, ae_ref):
    ve = jnp.dot(wedge_ref[...], attedge_ref[...],
                 preferred_element_type=jnp.float32)
    ea = jnp.maximum(jnp.dot(ea_ref[...], we_ref[...],
                             preferred_element_type=jnp.float32)
                     + be_ref[...], 0.0)
    ae_ref[...] = jnp.dot(ea, ve, preferred_element_type=jnp.float32)


def _tc_edge(edge_attr, We, be2, W_edge, att_edge2):
    be_ = 4000
    grid = E // be_
    return pl.pallas_call(
        _edge_body,
        grid=(grid,),
        in_specs=[
            pl.BlockSpec((be_, ED), lambda i: (i, 0)),
            pl.BlockSpec((ED, ED), lambda i: (0, 0)),
            pl.BlockSpec((1, ED), lambda i: (0, 0)),
            pl.BlockSpec((ED, F), lambda i: (0, 0)),
            pl.BlockSpec((F, 1), lambda i: (0, 0)),
        ],
        out_specs=pl.BlockSpec((be_, 1), lambda i: (i, 0)),
        out_shape=jax.ShapeDtypeStruct((E, 1), jnp.float32),
    )(edge_attr, We, be2, W_edge, att_edge2)


def _ln_body(p0_ref, p1_ref, x_ref, b_ref, g_ref, bt_ref, o_ref):
    out = p0_ref[...] + p1_ref[...] + x_ref[...] + b_ref[...]
    mu = jnp.mean(out, axis=-1, keepdims=True)
    cent = out - mu
    var = jnp.mean(cent * cent, axis=-1, keepdims=True)
    o_ref[...] = g_ref[...] * cent * lax.rsqrt(var + 1e-5) + bt_ref[...]


def _tc_ln(p0, p1, x, bias2, gamma2, beta2):
    br = 200
    grid = N // br
    vspec = pl.BlockSpec((1, F), lambda i: (0, 0))
    return pl.pallas_call(
        _ln_body,
        grid=(grid,),
        in_specs=[
            pl.BlockSpec((br, F), lambda i: (i, 0)),
            pl.BlockSpec((br, F), lambda i: (i, 0)),
            pl.BlockSpec((br, F), lambda i: (i, 0)),
            vspec, vspec, vspec,
        ],
        out_specs=pl.BlockSpec((br, F), lambda i: (i, 0)),
        out_shape=jax.ShapeDtypeStruct((N, F), jnp.float32),
    )(p0, p1, x, bias2, gamma2, beta2)


_GDN = lax.GatherDimensionNumbers(
    offset_dims=(), collapsed_slice_dims=(0,), start_index_map=(0,))


def _splat(vec, l):
    idx = jnp.full((L, 1), l, jnp.int32)
    return lax.gather(vec, idx, _GDN, slice_sizes=(1,),
                      mode=lax.GatherScatterMode.PROMISE_IN_BOUNDS)

def _sc_body(src_hbm, dst_hbm, ae_hbm, as_hbm, ad_hbm, h_hbm,
             out_hbm, ex_hbm,
             tab0, ad_t, src_c, dst_c, ae_c, ex_c,
             rows, sem,
             den_sh, out_sh):
    c = lax.axis_index("c")
    s = lax.axis_index("s")

    def _zero_rows(j, _):
        for v in range(F // L):
            rows[j, pl.ds(v * L, L)] = jnp.zeros((L,), jnp.float32)
        return 0
    lax.fori_loop(0, B, _zero_rows, 0)

    def _zero_tab(i, _):
        tab0[pl.ds(i * L, L)] = jnp.zeros((L,), jnp.float32)
        return 0
    lax.fori_loop(0, N // L, _zero_tab, 0)

    @pl.when(s == 0)
    def _():
        pltpu.sync_copy(tab0, den_sh)

    for i in range((NRC + NS - 1) // NS):
        k = s + NS * i

        @pl.when(k < NRC)
        def _():
            off = pl.multiple_of(k * B, 8)
            pltpu.sync_copy(rows, out_sh.at[pl.ds(off, B), :])

    plsc.subcore_barrier()

    pltpu.sync_copy(as_hbm, tab0)
    pltpu.sync_copy(ad_hbm, ad_t)

    mine = jnp.logical_or(jnp.logical_and(s < NS // 2, c == 0),
                          jnp.logical_and(s >= NS // 2, c == 1))

    for q in range(SPT):
        sb = SPT * s + q
        pltpu.sync_copy(src_hbm.at[sb], src_c)
        pltpu.sync_copy(dst_hbm.at[sb], dst_c)
        pltpu.sync_copy(ae_hbm.at[sb], ae_c)

        def _exbody(j, _):
            for g in range(B // L):
                sl = pl.ds(g * L, L)
                sidx = src_c[j, sl]
                didx = dst_c[j, sl]
                av = plsc.load_gather(tab0, [sidx])
                bv = plsc.load_gather(ad_t, [didx])
                e = av + bv + ae_c[j, sl]
                e = jnp.where(e >= 0.0, e, 0.2 * e)
                ex_c[j, sl] = jnp.exp(e)
            return 0
        lax.fori_loop(0, SB, _exbody, 0)

        def _denadd(j, _):
            pltpu.sync_copy(ex_c.at[j], den_sh.at[dst_c.at[j]], add=True)
            return 0
        lax.fori_loop(0, SB, _denadd, 0)

        @pl.when(mine)
        def _():
            pltpu.sync_copy(ex_c, ex_hbm.at[sb])

    plsc.subcore_barrier()

    pltpu.sync_copy(den_sh, tab0)
    m = c * NS + s

    for q in range(SPW):
        sb = SPW * m + q
        pltpu.sync_copy(src_hbm.at[sb], src_c)
        pltpu.sync_copy(dst_hbm.at[sb], dst_c)
        pltpu.sync_copy(ex_hbm.at[sb], ex_c)

        def _rowbody(j, _):
            cp = pltpu.async_copy(h_hbm.at[src_c.at[j]], rows, sem)
            for g in range(B // L):
                sl = pl.ds(g * L, L)
                didx = dst_c[j, sl]
                dv = plsc.load_gather(tab0, [didx])
                ex_c[j, sl] = ex_c[j, sl] / (dv + 1e-16)
            cp.wait()
            for g in range(B // L):
                a16 = ex_c[j, pl.ds(g * L, L)]

                def _lane(l, _):
                    sp = _splat(a16, l)
                    r = g * L + l
                    for v in range(F // L):
                        vs = pl.ds(v * L, L)
                        rows[r, vs] = rows[r, vs] * sp
                    return 0
                lax.fori_loop(0, L, _lane, 0)
            pltpu.sync_copy(rows, out_sh.at[dst_c.at[j]], add=True)
            return 0
        lax.fori_loop(0, SB, _rowbody, 0)

    plsc.subcore_barrier()

    for i in range((NRC + NS - 1) // NS):
        k = s + NS * i

        @pl.when(k < NRC)
        def _():
            off = pl.multiple_of(k * B, 8)
            rsl = pl.ds(off, B)
            pltpu.sync_copy(out_sh.at[rsl, :], rows)
            pltpu.sync_copy(rows, out_hbm.at[c].at[rsl, :])


def _sc_call(src2d, dst2d, ae2d, a_src, a_dst, h):
    mesh = plsc.VectorSubcoreMesh(core_axis_name="c", subcore_axis_name="s")
    fn = pl.kernel(
        _sc_body,
        out_type=[
            jax.ShapeDtypeStruct((NC, N, F), jnp.float32),
            jax.ShapeDtypeStruct((NSB, SB, B), jnp.float32),
        ],
        mesh=mesh,
        scratch_types=[
            pltpu.VMEM((N,), jnp.float32),
            pltpu.VMEM((N,), jnp.float32),
            pltpu.VMEM((SB, B), jnp.int32),
            pltpu.VMEM((SB, B), jnp.int32),
            pltpu.VMEM((SB, B), jnp.float32),
            pltpu.VMEM((SB, B), jnp.float32),
            pltpu.VMEM((B, F), jnp.float32),
            pltpu.SemaphoreType.DMA,
            pltpu.VMEM_SHARED((N,), jnp.float32),
            pltpu.VMEM_SHARED((N, F), jnp.float32),
        ],
        compiler_params=pltpu.CompilerParams(needs_layout_passes=False),
    )
    parts, _ = fn(src2d, dst2d, ae2d, a_src, a_dst, h)
    return parts


def kernel(x, edge_index, edge_attr, W, att_src, att_dst, W_edge, att_edge,
           bias, We, be, gamma, beta):
    av2 = jnp.concatenate(
        [att_src.reshape(F, 1), att_dst.reshape(F, 1)], axis=1)
    h, a2 = _tc_node(x, W, av2)
    a_src = a2[:, 0]
    a_dst = a2[:, 1]

    ae = _tc_edge(edge_attr, We, be.reshape(1, ED), W_edge,
                  att_edge.reshape(F, 1))
    ae2d = ae.reshape(NSB, SB, B)

    src2d = edge_index[0].reshape(NSB, SB, B)
    dst2d = edge_index[1].reshape(NSB, SB, B)

    parts = _sc_call(src2d, dst2d, ae2d, a_src, a_dst, h)

    return _tc_ln(parts[0], parts[1], x, bias.reshape(1, F),
                  gamma.reshape(1, F), beta.reshape(1, F))


# speedup vs baseline: 15.3764x; 1.1230x over previous
import functools

import jax
import jax.numpy as jnp
from jax import lax
from jax.experimental import pallas as pl
from jax.experimental.pallas import tpu as pltpu
from jax.experimental.pallas import tpu_sc as plsc

N = 10000
E = 320000
F = 128
ED = 16

NC = 2
NS = 16
L = 16

B = 80
SB = 25
NSB = E // (SB * B)
SPT = NSB // NS
SPW = NSB // (NC * NS)
NRC = N // B


def _node_body(x_ref, w_ref, as_ref, ad_ref, h_ref, asrc_ref, adst_ref):
    hb = jnp.dot(x_ref[...], w_ref[...], preferred_element_type=jnp.float32)
    h_ref[...] = hb
    asrc_ref[...] = jnp.sum(hb * as_ref[...], axis=1, keepdims=True)
    adst_ref[...] = jnp.sum(hb * ad_ref[...], axis=1, keepdims=True)


def _tc_node(x, W, att_src2, att_dst2):
    br = 1000
    grid = N // br
    vspec = pl.BlockSpec((1, F), lambda i: (0, 0))
    return pl.pallas_call(
        _node_body,
        grid=(grid,),
        in_specs=[
            pl.BlockSpec((br, F), lambda i: (i, 0)),
            pl.BlockSpec((F, F), lambda i: (0, 0)),
            vspec, vspec,
        ],
        out_specs=[
            pl.BlockSpec((br, F), lambda i: (i, 0)),
            pl.BlockSpec((br, 1), lambda i: (i, 0)),
            pl.BlockSpec((br, 1), lambda i: (i, 0)),
        ],
        out_shape=[
            jax.ShapeDtypeStruct((N, F), jnp.float32),
            jax.ShapeDtypeStruct((N, 1), jnp.float32),
            jax.ShapeDtypeStruct((N, 1), jnp.float32),
        ],
    )(x, W, att_src2, att_dst2)


def _edge_body(ea_ref, we_ref, be_ref, wedge_ref, attedge_ref, ae_ref):
    ve = jnp.dot(attedge_ref[...], wedge_ref[...],
                 preferred_element_type=jnp.float32)
    ea = jnp.maximum(jnp.dot(ea_ref[...], we_ref[...],
                             preferred_element_type=jnp.float32)
                     + be_ref[...], 0.0)
    ae_ref[...] = jnp.sum(ea * ve, axis=1, keepdims=True)


def _tc_edge(edge_attr, We, be2, W_edge_t, att_edge2):
    be_ = 8000
    grid = E // be_
    return pl.pallas_call(
        _edge_body,
        grid=(grid,),
        in_specs=[
            pl.BlockSpec((be_, ED), lambda i: (i, 0)),
            pl.BlockSpec((ED, ED), lambda i: (0, 0)),
            pl.BlockSpec((1, ED), lambda i: (0, 0)),
            pl.BlockSpec((F, ED), lambda i: (0, 0)),
            pl.BlockSpec((1, F), lambda i: (0, 0)),
        ],
        out_specs=pl.BlockSpec((be_, 1), lambda i: (i, 0)),
        out_shape=jax.ShapeDtypeStruct((E, 1), jnp.float32),
    )(edge_attr, We, be2, W_edge_t, att_edge2)


def _ln_body(p0_ref, p1_ref, x_ref, b_ref, g_ref, bt_ref, o_ref):
    out = p0_ref[...] + p1_ref[...] + x_ref[...] + b_ref[...]
    mu = jnp.mean(out, axis=-1, keepdims=True)
    cent = out - mu
    var = jnp.mean(cent * cent, axis=-1, keepdims=True)
    o_ref[...] = g_ref[...] * cent * lax.rsqrt(var + 1e-5) + bt_ref[...]


def _tc_ln(p0, p1, x, bias2, gamma2, beta2):
    br = 1000
    grid = N // br
    vspec = pl.BlockSpec((1, F), lambda i: (0, 0))
    return pl.pallas_call(
        _ln_body,
        grid=(grid,),
        in_specs=[
            pl.BlockSpec((br, F), lambda i: (i, 0)),
            pl.BlockSpec((br, F), lambda i: (i, 0)),
            pl.BlockSpec((br, F), lambda i: (i, 0)),
            vspec, vspec, vspec,
        ],
        out_specs=pl.BlockSpec((br, F), lambda i: (i, 0)),
        out_shape=jax.ShapeDtypeStruct((N, F), jnp.float32),
    )(p0, p1, x, bias2, gamma2, beta2)


_GDN = lax.GatherDimensionNumbers(
    offset_dims=(), collapsed_slice_dims=(0,), start_index_map=(0,))


def _splat(vec, l):
    idx = jnp.full((L, 1), l, jnp.int32)
    return lax.gather(vec, idx, _GDN, slice_sizes=(1,),
                      mode=lax.GatherScatterMode.PROMISE_IN_BOUNDS)

def _sc_body(ei_hbm, ae_hbm, as_hbm, ad_hbm, h_hbm,
             out_hbm, ex_hbm,
             tab0, ad_t, src_c, dst_c, ae_c, ex_c,
             rows, sem,
             den_sh, out_sh):
    c = lax.axis_index("c")
    s = lax.axis_index("s")

    def _zero_rows(j, _):
        for v in range(F // L):
            rows[j, pl.ds(v * L, L)] = jnp.zeros((L,), jnp.float32)
        return 0
    lax.fori_loop(0, B, _zero_rows, 0)

    def _zero_tab(i, _):
        tab0[pl.ds(i * L, L)] = jnp.zeros((L,), jnp.float32)
        return 0
    lax.fori_loop(0, N // L, _zero_tab, 0)

    @pl.when(s == 0)
    def _():
        pltpu.sync_copy(tab0, den_sh)

    for i in range((NRC + NS - 1) // NS):
        k = s + NS * i

        @pl.when(k < NRC)
        def _():
            off = pl.multiple_of(k * B, 8)
            pltpu.sync_copy(rows, out_sh.at[pl.ds(off, B), :])

    plsc.subcore_barrier()

    pltpu.sync_copy(as_hbm, tab0)
    pltpu.sync_copy(ad_hbm, ad_t)

    mine = jnp.logical_or(jnp.logical_and(s < NS // 2, c == 0),
                          jnp.logical_and(s >= NS // 2, c == 1))

    for q in range(SPT):
        sb = SPT * s + q
        pltpu.sync_copy(ei_hbm.at[0, sb], src_c)
        pltpu.sync_copy(ei_hbm.at[1, sb], dst_c)
        pltpu.sync_copy(ae_hbm.at[sb], ae_c)

        def _exbody(j, _):
            for g in range(B // L):
                sl = pl.ds(g * L, L)
                sidx = src_c[j, sl]
                didx = dst_c[j, sl]
                av = plsc.load_gather(tab0, [sidx])
                bv = plsc.load_gather(ad_t, [didx])
                e = av + bv + ae_c[j, sl]
                e = jnp.where(e >= 0.0, e, 0.2 * e)
                ex_c[j, sl] = jnp.exp(e)
            return 0
        lax.fori_loop(0, SB, _exbody, 0)

        def _denadd(j, _):
            pltpu.sync_copy(ex_c.at[j], den_sh.at[dst_c.at[j]], add=True)
            return 0
        lax.fori_loop(0, SB, _denadd, 0)

        @pl.when(mine)
        def _():
            pltpu.sync_copy(ex_c, ex_hbm.at[sb])

    plsc.subcore_barrier()

    pltpu.sync_copy(den_sh, tab0)
    m = c * NS + s

    for q in range(SPW):
        sb = SPW * m + q
        pltpu.sync_copy(ei_hbm.at[0, sb], src_c)
        pltpu.sync_copy(ei_hbm.at[1, sb], dst_c)
        pltpu.sync_copy(ex_hbm.at[sb], ex_c)

        def _rowbody(j, _):
            cp = pltpu.async_copy(h_hbm.at[src_c.at[j]], rows, sem)
            for g in range(B // L):
                sl = pl.ds(g * L, L)
                didx = dst_c[j, sl]
                dv = plsc.load_gather(tab0, [didx])
                ex_c[j, sl] = ex_c[j, sl] / (dv + 1e-16)
            cp.wait()
            for g in range(B // L):
                a16 = ex_c[j, pl.ds(g * L, L)]

                def _lane(l, _):
                    sp = _splat(a16, l)
                    r = g * L + l
                    for v in range(F // L):
                        vs = pl.ds(v * L, L)
                        rows[r, vs] = rows[r, vs] * sp
                    return 0
                lax.fori_loop(0, L, _lane, 0)
            pltpu.sync_copy(rows, out_sh.at[dst_c.at[j]], add=True)
            return 0
        lax.fori_loop(0, SB, _rowbody, 0)

    plsc.subcore_barrier()

    for i in range((NRC + NS - 1) // NS):
        k = s + NS * i

        @pl.when(k < NRC)
        def _():
            off = pl.multiple_of(k * B, 8)
            rsl = pl.ds(off, B)
            pltpu.sync_copy(out_sh.at[rsl, :], rows)
            pltpu.sync_copy(rows, out_hbm.at[c].at[rsl, :])


def _sc_call(ei4, ae2d, a_src, a_dst, h):
    mesh = plsc.VectorSubcoreMesh(core_axis_name="c", subcore_axis_name="s")
    fn = pl.kernel(
        _sc_body,
        out_type=[
            jax.ShapeDtypeStruct((NC, N, F), jnp.float32),
            jax.ShapeDtypeStruct((NSB, SB, B), jnp.float32),
        ],
        mesh=mesh,
        scratch_types=[
            pltpu.VMEM((N,), jnp.float32),
            pltpu.VMEM((N,), jnp.float32),
            pltpu.VMEM((SB, B), jnp.int32),
            pltpu.VMEM((SB, B), jnp.int32),
            pltpu.VMEM((SB, B), jnp.float32),
            pltpu.VMEM((SB, B), jnp.float32),
            pltpu.VMEM((B, F), jnp.float32),
            pltpu.SemaphoreType.DMA,
            pltpu.VMEM_SHARED((N,), jnp.float32),
            pltpu.VMEM_SHARED((N, F), jnp.float32),
        ],
        compiler_params=pltpu.CompilerParams(needs_layout_passes=False),
    )
    parts, _ = fn(ei4, ae2d, a_src, a_dst, h)
    return parts


def kernel(x, edge_index, edge_attr, W, att_src, att_dst, W_edge, att_edge,
           bias, We, be, gamma, beta):
    h, asrc, adst = _tc_node(x, W, att_src.reshape(1, F),
                             att_dst.reshape(1, F))
    a_src = asrc.reshape(N)
    a_dst = adst.reshape(N)

    ae = _tc_edge(edge_attr, We, be.reshape(1, ED), W_edge.T,
                  att_edge.reshape(1, F))
    ae2d = ae.reshape(NSB, SB, B)

    ei4 = edge_index.reshape(2, NSB, SB, B)

    parts = _sc_call(ei4, ae2d, a_src, a_dst, h)

    return _tc_ln(parts[0], parts[1], x, bias.reshape(1, F),
                  gamma.reshape(1, F), beta.reshape(1, F))


# speedup vs baseline: 22.5992x; 1.4697x over previous
import functools

import jax
import jax.numpy as jnp
from jax import lax
from jax.experimental import pallas as pl
from jax.experimental.pallas import tpu as pltpu
from jax.experimental.pallas import tpu_sc as plsc

N = 10000
E = 320000
F = 128
ED = 16

NC = 2
NS = 16
L = 16

B = 80
SB = 25
NSB = E // (SB * B)
SPT = NSB // NS
SPW = NSB // (NC * NS)
NRC = N // B


def _node_body(x_ref, w_ref, as_ref, ad_ref, h_ref, asrc_ref, adst_ref):
    hb = jnp.dot(x_ref[...], w_ref[...], preferred_element_type=jnp.float32)
    h_ref[...] = hb
    asrc_ref[...] = jnp.sum(hb * as_ref[...], axis=1, keepdims=True)
    adst_ref[...] = jnp.sum(hb * ad_ref[...], axis=1, keepdims=True)


def _tc_node(x, W, att_src2, att_dst2):
    br = 1000
    grid = N // br
    vspec = pl.BlockSpec((1, F), lambda i: (0, 0))
    return pl.pallas_call(
        _node_body,
        grid=(grid,),
        in_specs=[
            pl.BlockSpec((br, F), lambda i: (i, 0)),
            pl.BlockSpec((F, F), lambda i: (0, 0)),
            vspec, vspec,
        ],
        out_specs=[
            pl.BlockSpec((br, F), lambda i: (i, 0)),
            pl.BlockSpec((br, 1), lambda i: (i, 0)),
            pl.BlockSpec((br, 1), lambda i: (i, 0)),
        ],
        out_shape=[
            jax.ShapeDtypeStruct((N, F), jnp.float32),
            jax.ShapeDtypeStruct((N, 1), jnp.float32),
            jax.ShapeDtypeStruct((N, 1), jnp.float32),
        ],
    )(x, W, att_src2, att_dst2)


def _edge_body(eat_ref, wet_ref, be_ref, wedge_ref, attedge_ref, ae_ref):
    ve = jnp.dot(wedge_ref[...], attedge_ref[...],
                 preferred_element_type=jnp.float32)
    ea = jnp.maximum(jnp.dot(wet_ref[...], eat_ref[...],
                             preferred_element_type=jnp.float32)
                     + be_ref[...], 0.0)
    ae_ref[...] = jnp.sum(ea * ve, axis=0, keepdims=True)


def _tc_edge(edge_attr_t, We_t, be2, W_edge, att_edge2):
    be_ = 16000
    grid = E // be_
    return pl.pallas_call(
        _edge_body,
        grid=(grid,),
        in_specs=[
            pl.BlockSpec((ED, be_), lambda i: (0, i)),
            pl.BlockSpec((ED, ED), lambda i: (0, 0)),
            pl.BlockSpec((ED, 1), lambda i: (0, 0)),
            pl.BlockSpec((ED, F), lambda i: (0, 0)),
            pl.BlockSpec((F, 1), lambda i: (0, 0)),
        ],
        out_specs=pl.BlockSpec((1, be_), lambda i: (0, i)),
        out_shape=jax.ShapeDtypeStruct((1, E), jnp.float32),
    )(edge_attr_t, We_t, be2, W_edge, att_edge2)


def _ln_body(p0_ref, p1_ref, x_ref, b_ref, g_ref, bt_ref, o_ref):
    out = p0_ref[...] + p1_ref[...] + x_ref[...] + b_ref[...]
    mu = jnp.mean(out, axis=-1, keepdims=True)
    cent = out - mu
    var = jnp.mean(cent * cent, axis=-1, keepdims=True)
    o_ref[...] = g_ref[...] * cent * lax.rsqrt(var + 1e-5) + bt_ref[...]


def _tc_ln(p0, p1, x, bias2, gamma2, beta2):
    br = 1000
    grid = N // br
    vspec = pl.BlockSpec((1, F), lambda i: (0, 0))
    return pl.pallas_call(
        _ln_body,
        grid=(grid,),
        in_specs=[
            pl.BlockSpec((br, F), lambda i: (i, 0)),
            pl.BlockSpec((br, F), lambda i: (i, 0)),
            pl.BlockSpec((br, F), lambda i: (i, 0)),
            vspec, vspec, vspec,
        ],
        out_specs=pl.BlockSpec((br, F), lambda i: (i, 0)),
        out_shape=jax.ShapeDtypeStruct((N, F), jnp.float32),
    )(p0, p1, x, bias2, gamma2, beta2)


_GDN = lax.GatherDimensionNumbers(
    offset_dims=(), collapsed_slice_dims=(0,), start_index_map=(0,))


def _splat(vec, l):
    idx = jnp.full((L, 1), l, jnp.int32)
    return lax.gather(vec, idx, _GDN, slice_sizes=(1,),
                      mode=lax.GatherScatterMode.PROMISE_IN_BOUNDS)

def _sc_body(ei_hbm, ae_hbm, as_hbm, ad_hbm, h_hbm,
             out_hbm, ex_hbm,
             tab0, ad_t, src_c, dst_c, ae_c, ex_c,
             rows, sem,
             den_sh, out_sh):
    c = lax.axis_index("c")
    s = lax.axis_index("s")

    def _zero_rows(j, _):
        for v in range(F // L):
            rows[j, pl.ds(v * L, L)] = jnp.zeros((L,), jnp.float32)
        return 0
    lax.fori_loop(0, B, _zero_rows, 0)

    def _zero_tab(i, _):
        tab0[pl.ds(i * L, L)] = jnp.zeros((L,), jnp.float32)
        return 0
    lax.fori_loop(0, N // L, _zero_tab, 0)

    @pl.when(s == 0)
    def _():
        pltpu.sync_copy(tab0, den_sh)

    for i in range((NRC + NS - 1) // NS):
        k = s + NS * i

        @pl.when(k < NRC)
        def _():
            off = pl.multiple_of(k * B, 8)
            pltpu.sync_copy(rows, out_sh.at[pl.ds(off, B), :])

    plsc.subcore_barrier()

    pltpu.sync_copy(as_hbm, tab0)
    pltpu.sync_copy(ad_hbm, ad_t)

    mine = jnp.logical_or(jnp.logical_and(s < NS // 2, c == 0),
                          jnp.logical_and(s >= NS // 2, c == 1))

    for q in range(SPT):
        sb = SPT * s + q
        pltpu.sync_copy(ei_hbm.at[0, sb], src_c)
        pltpu.sync_copy(ei_hbm.at[1, sb], dst_c)
        pltpu.sync_copy(ae_hbm.at[pl.ds(sb * SB * B, SB * B)], ae_c)

        def _exbody(j, _):
            for g in range(B // L):
                sl = pl.ds(g * L, L)
                sidx = src_c[j, sl]
                didx = dst_c[j, sl]
                av = plsc.load_gather(tab0, [sidx])
                bv = plsc.load_gather(ad_t, [didx])
                e = av + bv + ae_c[pl.ds(j * B + g * L, L)]
                e = jnp.where(e >= 0.0, e, 0.2 * e)
                ex_c[j, sl] = jnp.exp(e)
            return 0
        lax.fori_loop(0, SB, _exbody, 0)

        def _denadd(j, _):
            pltpu.sync_copy(ex_c.at[j], den_sh.at[dst_c.at[j]], add=True)
            return 0
        lax.fori_loop(0, SB, _denadd, 0)

        @pl.when(mine)
        def _():
            pltpu.sync_copy(ex_c, ex_hbm.at[sb])

    plsc.subcore_barrier()

    pltpu.sync_copy(den_sh, tab0)
    m = c * NS + s

    for q in range(SPW):
        sb = SPW * m + q
        pltpu.sync_copy(ei_hbm.at[0, sb], src_c)
        pltpu.sync_copy(ei_hbm.at[1, sb], dst_c)
        pltpu.sync_copy(ex_hbm.at[sb], ex_c)

        def _rowbody(j, _):
            cp = pltpu.async_copy(h_hbm.at[src_c.at[j]], rows, sem)
            for g in range(B // L):
                sl = pl.ds(g * L, L)
                didx = dst_c[j, sl]
                dv = plsc.load_gather(tab0, [didx])
                ex_c[j, sl] = ex_c[j, sl] / (dv + 1e-16)
            cp.wait()
            for g in range(B // L):
                a16 = ex_c[j, pl.ds(g * L, L)]

                def _lane(l, _):
                    sp = _splat(a16, l)
                    r = g * L + l
                    for v in range(F // L):
                        vs = pl.ds(v * L, L)
                        rows[r, vs] = rows[r, vs] * sp
                    return 0
                lax.fori_loop(0, L, _lane, 0)
            pltpu.sync_copy(rows, out_sh.at[dst_c.at[j]], add=True)
            return 0
        lax.fori_loop(0, SB, _rowbody, 0)

    plsc.subcore_barrier()

    for i in range((NRC + NS - 1) // NS):
        k = s + NS * i

        @pl.when(k < NRC)
        def _():
            off = pl.multiple_of(k * B, 8)
            rsl = pl.ds(off, B)
            pltpu.sync_copy(out_sh.at[rsl, :], rows)
            pltpu.sync_copy(rows, out_hbm.at[c].at[rsl, :])


def _sc_call(ei4, ae2d, a_src, a_dst, h):
    mesh = plsc.VectorSubcoreMesh(core_axis_name="c", subcore_axis_name="s")
    fn = pl.kernel(
        _sc_body,
        out_type=[
            jax.ShapeDtypeStruct((NC, N, F), jnp.float32),
            jax.ShapeDtypeStruct((NSB, SB, B), jnp.float32),
        ],
        mesh=mesh,
        scratch_types=[
            pltpu.VMEM((N,), jnp.float32),
            pltpu.VMEM((N,), jnp.float32),
            pltpu.VMEM((SB, B), jnp.int32),
            pltpu.VMEM((SB, B), jnp.int32),
            pltpu.VMEM((SB * B,), jnp.float32),
            pltpu.VMEM((SB, B), jnp.float32),
            pltpu.VMEM((B, F), jnp.float32),
            pltpu.SemaphoreType.DMA,
            pltpu.VMEM_SHARED((N,), jnp.float32),
            pltpu.VMEM_SHARED((N, F), jnp.float32),
        ],
        compiler_params=pltpu.CompilerParams(needs_layout_passes=False),
    )
    parts, _ = fn(ei4, ae2d, a_src, a_dst, h)
    return parts


def kernel(x, edge_index, edge_attr, W, att_src, att_dst, W_edge, att_edge,
           bias, We, be, gamma, beta):
    h, asrc, adst = _tc_node(x, W, att_src.reshape(1, F),
                             att_dst.reshape(1, F))
    a_src = asrc.reshape(N)
    a_dst = adst.reshape(N)

    ae = _tc_edge(edge_attr.T, We.T, be.reshape(ED, 1), W_edge,
                  att_edge.reshape(F, 1))
    ae1 = ae.reshape(E)

    ei4 = edge_index.reshape(2, NSB, SB, B)

    parts = _sc_call(ei4, ae1, a_src, a_dst, h)

    return _tc_ln(parts[0], parts[1], x, bias.reshape(1, F),
                  gamma.reshape(1, F), beta.reshape(1, F))


# speedup vs baseline: 29.0950x; 1.2874x over previous
import functools

import jax
import jax.numpy as jnp
from jax import lax
from jax.experimental import pallas as pl
from jax.experimental.pallas import tpu as pltpu
from jax.experimental.pallas import tpu_sc as plsc

N = 10000
E = 320000
F = 128
ED = 16

NC = 2
NS = 16
L = 16

B = 80
SB = 25
NSB = E // (SB * B)
SPT = NSB // NS
SPW = NSB // (NC * NS)
NRC = N // B


def _node_body(x_ref, w_ref, as_ref, ad_ref, h_ref, asrc_ref, adst_ref):
    hb = jnp.dot(x_ref[...], w_ref[...], preferred_element_type=jnp.float32)
    h_ref[...] = hb
    asrc_ref[...] = jnp.sum(hb * as_ref[...], axis=1, keepdims=True)
    adst_ref[...] = jnp.sum(hb * ad_ref[...], axis=1, keepdims=True)


def _tc_node(x, W, att_src2, att_dst2):
    br = 1000
    grid = N // br
    vspec = pl.BlockSpec((1, F), lambda i: (0, 0))
    return pl.pallas_call(
        _node_body,
        grid=(grid,),
        in_specs=[
            pl.BlockSpec((br, F), lambda i: (i, 0)),
            pl.BlockSpec((F, F), lambda i: (0, 0)),
            vspec, vspec,
        ],
        out_specs=[
            pl.BlockSpec((br, F), lambda i: (i, 0)),
            pl.BlockSpec((br, 1), lambda i: (i, 0)),
            pl.BlockSpec((br, 1), lambda i: (i, 0)),
        ],
        out_shape=[
            jax.ShapeDtypeStruct((N, F), jnp.float32),
            jax.ShapeDtypeStruct((N, 1), jnp.float32),
            jax.ShapeDtypeStruct((N, 1), jnp.float32),
        ],
    )(x, W, att_src2, att_dst2)


def _edge_body(eat_ref, wet_ref, be_ref, wedge_ref, attedge_ref, ae_ref):
    ve = jnp.dot(wedge_ref[...], attedge_ref[...],
                 preferred_element_type=jnp.float32)
    ea = jnp.maximum(jnp.dot(wet_ref[...], eat_ref[...],
                             preferred_element_type=jnp.float32)
                     + be_ref[...], 0.0)
    ae_ref[...] = jnp.sum(ea * ve, axis=0, keepdims=True)


def _tc_edge(edge_attr_t, We_t, be2, W_edge, att_edge2):
    be_ = 16000
    grid = E // be_
    return pl.pallas_call(
        _edge_body,
        grid=(grid,),
        in_specs=[
            pl.BlockSpec((ED, be_), lambda i: (0, i)),
            pl.BlockSpec((ED, ED), lambda i: (0, 0)),
            pl.BlockSpec((ED, 1), lambda i: (0, 0)),
            pl.BlockSpec((ED, F), lambda i: (0, 0)),
            pl.BlockSpec((F, 1), lambda i: (0, 0)),
        ],
        out_specs=pl.BlockSpec((1, be_), lambda i: (0, i)),
        out_shape=jax.ShapeDtypeStruct((1, E), jnp.float32),
    )(edge_attr_t, We_t, be2, W_edge, att_edge2)


def _ln_body(p0_ref, p1_ref, x_ref, b_ref, g_ref, bt_ref, o_ref):
    out = p0_ref[...] + p1_ref[...] + x_ref[...] + b_ref[...]
    mu = jnp.mean(out, axis=-1, keepdims=True)
    cent = out - mu
    var = jnp.mean(cent * cent, axis=-1, keepdims=True)
    o_ref[...] = g_ref[...] * cent * lax.rsqrt(var + 1e-5) + bt_ref[...]


def _tc_ln(p0, p1, x, bias2, gamma2, beta2):
    br = 1000
    grid = N // br
    vspec = pl.BlockSpec((1, F), lambda i: (0, 0))
    return pl.pallas_call(
        _ln_body,
        grid=(grid,),
        in_specs=[
            pl.BlockSpec((br, F), lambda i: (i, 0)),
            pl.BlockSpec((br, F), lambda i: (i, 0)),
            pl.BlockSpec((br, F), lambda i: (i, 0)),
            vspec, vspec, vspec,
        ],
        out_specs=pl.BlockSpec((br, F), lambda i: (i, 0)),
        out_shape=jax.ShapeDtypeStruct((N, F), jnp.float32),
    )(p0, p1, x, bias2, gamma2, beta2)


_GDN = lax.GatherDimensionNumbers(
    offset_dims=(), collapsed_slice_dims=(0,), start_index_map=(0,))


def _splat(vec, l):
    idx = jnp.full((L, 1), l, jnp.int32)
    return lax.gather(vec, idx, _GDN, slice_sizes=(1,),
                      mode=lax.GatherScatterMode.PROMISE_IN_BOUNDS)

def _sc_body(ei_hbm, ae_hbm, as_hbm, ad_hbm, h_hbm,
             out_hbm, ex_hbm,
             tab0, adu, src_c, dst_c, ae_c, ex_c,
             rows, sem, sem_b, sem_d,
             den_sh, out_sh):
    c = lax.axis_index("c")
    s = lax.axis_index("s")

    def _zero_rows(j, _):
        for v in range(F // L):
            rows[j, pl.ds(v * L, L)] = jnp.zeros((L,), jnp.float32)
        return 0
    lax.fori_loop(0, B, _zero_rows, 0)

    def _zero_tab(i, _):
        tab0[pl.ds(i * L, L)] = jnp.zeros((L,), jnp.float32)
        return 0
    lax.fori_loop(0, N // L, _zero_tab, 0)

    @pl.when(s == 0)
    def _():
        pltpu.sync_copy(tab0, den_sh)

    for i in range((NRC + NS - 1) // NS):
        k = s + NS * i

        @pl.when(k < NRC)
        def _():
            off = pl.multiple_of(k * B, 8)
            pltpu.sync_copy(rows, out_sh.at[pl.ds(off, B), :])

    plsc.subcore_barrier()

    pltpu.sync_copy(as_hbm, tab0)
    pltpu.sync_copy(ad_hbm, adu)

    mine = jnp.logical_or(jnp.logical_and(s < NS // 2, c == 0),
                          jnp.logical_and(s >= NS // 2, c == 1))

    def _p1(q, _):
        sb = SPT * s + q
        pltpu.sync_copy(ei_hbm.at[0, sb], src_c)
        pltpu.sync_copy(ei_hbm.at[1, sb], dst_c)
        pltpu.sync_copy(ae_hbm.at[pl.ds(sb * SB * B, SB * B)], ae_c)

        def _exbody(j, _):
            for g in range(B // L):
                sl = pl.ds(g * L, L)
                sidx = src_c[j, sl]
                didx = dst_c[j, sl]
                av = plsc.load_gather(tab0, [sidx])
                bv = plsc.load_gather(
                    adu, [lax.shift_right_logical(didx, 7),
                          jnp.bitwise_and(didx, 127)])
                e = av + bv + ae_c[pl.ds(j * B + g * L, L)]
                e = jnp.where(e >= 0.0, e, 0.2 * e)
                ex_c[j, sl] = jnp.exp(e)
            return 0
        lax.fori_loop(0, SB, _exbody, 0)

        def _denadd(j, _):
            pltpu.async_copy(ex_c.at[j], den_sh.at[dst_c.at[j]], sem_d,
                             add=True)
            return 0
        lax.fori_loop(0, SB, _denadd, 0)

        @pl.when(mine)
        def _():
            pltpu.sync_copy(ex_c, ex_hbm.at[sb])

        def _dendrain(j, _):
            pltpu.make_async_copy(ex_c.at[j], den_sh.at[dst_c.at[j]],
                                  sem_d).wait()
            return 0
        lax.fori_loop(0, SB, _dendrain, 0)
        return 0
    lax.fori_loop(0, SPT, _p1, 0)

    plsc.subcore_barrier()

    pltpu.sync_copy(den_sh, tab0)
    m = c * NS + s

    def _scale(rref, jb):
        for g in range(B // L):
            a16 = ex_c[jb, pl.ds(g * L, L)]
            for l in range(L):
                sp = _splat(a16, l)
                r = g * L + l
                for v in range(F // L):
                    vs = pl.ds(v * L, L)
                    rref[r, vs] = rref[r, vs] * sp

    def _p2(q, _):
        sb = SPW * m + q
        pltpu.sync_copy(ei_hbm.at[0, sb], src_c)
        pltpu.sync_copy(ei_hbm.at[1, sb], dst_c)
        pltpu.sync_copy(ex_hbm.at[sb], ex_c)

        pltpu.async_copy(h_hbm.at[src_c.at[0]], rows, sem)

        def _alpha(j, _):
            for g in range(B // L):
                sl = pl.ds(g * L, L)
                didx = dst_c[j, sl]
                dv = plsc.load_gather(tab0, [didx])
                ex_c[j, sl] = ex_c[j, sl] / (dv + 1e-16)
            return 0
        lax.fori_loop(0, SB, _alpha, 0)

        def _pair(i, _):
            j0 = 2 * i

            @pl.when(j0 + 1 < SB)
            def _():
                pltpu.async_copy(h_hbm.at[src_c.at[j0 + 1]], adu, sem_b)

            pltpu.make_async_copy(h_hbm.at[src_c.at[j0]], rows, sem).wait()
            _scale(rows, j0)
            pltpu.sync_copy(rows, out_sh.at[dst_c.at[j0]], add=True)

            @pl.when(j0 + 2 < SB)
            def _():
                pltpu.async_copy(h_hbm.at[src_c.at[j0 + 2]], rows, sem)

            @pl.when(j0 + 1 < SB)
            def _():
                pltpu.make_async_copy(h_hbm.at[src_c.at[j0 + 1]], adu,
                                      sem_b).wait()
                _scale(adu, j0 + 1)
                pltpu.sync_copy(adu, out_sh.at[dst_c.at[j0 + 1]], add=True)
            return 0
        lax.fori_loop(0, (SB + 1) // 2, _pair, 0)
        return 0
    lax.fori_loop(0, SPW, _p2, 0)

    plsc.subcore_barrier()

    for i in range((NRC + NS - 1) // NS):
        k = s + NS * i

        @pl.when(k < NRC)
        def _():
            off = pl.multiple_of(k * B, 8)
            rsl = pl.ds(off, B)
            pltpu.sync_copy(out_sh.at[rsl, :], rows)
            pltpu.sync_copy(rows, out_hbm.at[c].at[rsl, :])


def _sc_call(ei4, ae2d, a_src, a_dst, h):
    mesh = plsc.VectorSubcoreMesh(core_axis_name="c", subcore_axis_name="s")
    fn = pl.kernel(
        _sc_body,
        out_type=[
            jax.ShapeDtypeStruct((NC, N, F), jnp.float32),
            jax.ShapeDtypeStruct((NSB, SB, B), jnp.float32),
        ],
        mesh=mesh,
        scratch_types=[
            pltpu.VMEM((N,), jnp.float32),
            pltpu.VMEM((B, F), jnp.float32),
            pltpu.VMEM((SB, B), jnp.int32),
            pltpu.VMEM((SB, B), jnp.int32),
            pltpu.VMEM((SB * B,), jnp.float32),
            pltpu.VMEM((SB, B), jnp.float32),
            pltpu.VMEM((B, F), jnp.float32),
            pltpu.SemaphoreType.DMA,
            pltpu.SemaphoreType.DMA,
            pltpu.SemaphoreType.DMA,
            pltpu.VMEM_SHARED((N,), jnp.float32),
            pltpu.VMEM_SHARED((N, F), jnp.float32),
        ],
        compiler_params=pltpu.CompilerParams(needs_layout_passes=False),
    )
    parts, _ = fn(ei4, ae2d, a_src, a_dst, h)
    return parts


def kernel(x, edge_index, edge_attr, W, att_src, att_dst, W_edge, att_edge,
           bias, We, be, gamma, beta):
    h, asrc, adst = _tc_node(x, W, att_src.reshape(1, F),
                             att_dst.reshape(1, F))
    a_src = asrc.reshape(N)
    a_dst2d = jnp.pad(adst.reshape(N), (0, B * F - N)).reshape(B, F)

    ae = _tc_edge(edge_attr.T, We.T, be.reshape(ED, 1), W_edge,
                  att_edge.reshape(F, 1))
    ae1 = ae.reshape(E)

    ei4 = edge_index.reshape(2, NSB, SB, B)

    parts = _sc_call(ei4, ae1, a_src, a_dst2d, h)

    return _tc_ln(parts[0], parts[1], x, bias.reshape(1, F),
                  gamma.reshape(1, F), beta.reshape(1, F))
